# baseline (speedup 1.0000x reference)
import functools

import jax
import jax.numpy as jnp
import numpy as np
from jax import lax
from jax.experimental import pallas as pl
from jax.experimental.pallas import tpu as pltpu
from jax.experimental.pallas import tpu_sc as plsc

N = 10000
E = 640000
IN_C = 116
HID = 128
OUT_C = 2

NC, NS = 2, 16
NW = NC * NS
EPW = E // NW
B = 80
NWIN = EPW // B
NBUF = 5
CW = 50
ROWS_BIG = 640
ROWS_LAST = N - 15 * ROWS_BIG
ZR = 80

BR = 2000
GR = N // BR

TAU = np.concatenate(
    [np.concatenate([np.arange(b * 32, b * 32 + 32, 2),
                     np.arange(b * 32 + 1, b * 32 + 32, 2)])
     for b in range(HID // 32)])

@functools.lru_cache(maxsize=None)
def _mesh():
  return plsc.VectorSubcoreMesh(
      core_axis_name="c", subcore_axis_name="s", num_cores=NC, num_subcores=NS)


def _zero_fill(zbuf, F):
  zv = jnp.zeros((16,), jnp.float32)

  @pl.loop(0, ZR)
  def _(r):
    for j in range(F // 16):
      zbuf[r, pl.ds(j * 16, 16)] = zv


def _zero_acc(zbuf, acc, s, F):
  _zero_fill(zbuf, F)
  nz = jnp.where(s == NS - 1, ROWS_LAST // ZR, ROWS_BIG // ZR)

  @pl.loop(0, nz)
  def _(k):
    pltpu.sync_copy(zbuf, acc.at[pl.ds(s * ROWS_BIG + k * ZR, ZR)])


def _readout(acc, out_hbm, c, s):
  @pl.when(s < NS - 1)
  def _():
    pltpu.sync_copy(acc.at[pl.ds(s * ROWS_BIG, ROWS_BIG)],
                    out_hbm.at[c, pl.ds(s * ROWS_BIG, ROWS_BIG)])

  @pl.when(s == NS - 1)
  def _():
    pltpu.sync_copy(acc.at[pl.ds((NS - 1) * ROWS_BIG, ROWS_LAST)],
                    out_hbm.at[c, pl.ds((NS - 1) * ROWS_BIG, ROWS_LAST)])


@functools.lru_cache(maxsize=None)
def _make_spmm(feature_split):
  if feature_split:
    fa = HID // NC
    nwin = E // NS // B
    cw, nbuf = 100, 5
    gdt = jnp.bfloat16
  else:
    fa = 16
    nwin = E // NW // B
    cw, nbuf = 125, 5
    gdt = jnp.float32
  nch = nwin // cw
  ngrp = cw // nbuf
  bf16 = gdt == jnp.bfloat16

  scratch = [
      pltpu.VMEM((2, cw, B), jnp.int32),
      pltpu.VMEM((2, cw, B), jnp.int32),
      pltpu.VMEM_SHARED((N, fa), jnp.float32),
      pltpu.VMEM((ZR, fa), jnp.float32),
  ]
  scratch += [pltpu.VMEM((B, fa), jnp.float32) for _ in range(nbuf)]
  if bf16:
    scratch += [pltpu.VMEM((B, fa), gdt) for _ in range(nbuf)]
  scratch += [pltpu.SemaphoreType.DMA for _ in range(2 * nbuf + 1)]

  @functools.partial(
      pl.kernel,
      out_type=jax.ShapeDtypeStruct((NC, N, fa), jnp.float32),
      mesh=_mesh(),
      compiler_params=pltpu.CompilerParams(use_tc_tiling_on_sc=False,
                                           needs_layout_passes=not bf16),
      scratch_types=scratch)
  def spmm(hs_hbm, src_hbm, dst_hbm, out_hbm, srcv, dstv, acc, zbuf, *rest):
    rows = rest[:nbuf]
    rest = rest[nbuf:]
    if bf16:
      brows = rest[:nbuf]
      rest = rest[nbuf:]
    gsem = rest[:nbuf]
    ssem = rest[nbuf:2 * nbuf]
    isem = rest[2 * nbuf]
    c = lax.axis_index("c")
    s = lax.axis_index("s")
    iw = s if feature_split else c * NS + s
    table = hs_hbm.at[c] if feature_split else hs_hbm
    gbuf = brows if bf16 else rows

    def convert(b):
      if not bf16:
        return

      @pl.loop(0, B, unroll=8)
      def _(r):
        for j in range(fa // 32):
          ev, od = plsc.unpack(brows[b][r, pl.ds(j * 32, 32)],
                               format=plsc.PackFormat.INTERLEAVED)
          rows[b][r, pl.ds(j * 32, 16)] = ev
          rows[b][r, pl.ds(j * 32 + 16, 16)] = od

    def ichunk_start(k, p):
      pltpu.async_copy(src_hbm.at[iw, pl.ds(k * cw, cw)], srcv.at[p], isem)
      pltpu.async_copy(dst_hbm.at[iw, pl.ds(k * cw, cw)], dstv.at[p], isem)

    def ichunk_wait(p):
      pltpu.make_async_copy(src_hbm.at[iw, pl.ds(0, cw)], srcv.at[p],
                            isem).wait()
      pltpu.make_async_copy(dst_hbm.at[iw, pl.ds(0, cw)], dstv.at[p],
                            isem).wait()

    def gather_start(p, w, b):
      pltpu.async_copy(table.at[srcv.at[p, w]], gbuf[b], gsem[b])

    def gather_wait(b):
      pltpu.make_async_copy(table.at[srcv.at[0, 0]], gbuf[b], gsem[b]).wait()

    def scat_start(p, w, b):
      pltpu.async_copy(rows[b], acc.at[dstv.at[p, w]], ssem[b], add=True)

    def scat_wait(b):
      pltpu.make_async_copy(rows[b], acc.at[dstv.at[0, 0]], ssem[b]).wait()

    ichunk_start(0, 0)
    _zero_acc(zbuf, acc, s, fa)
    plsc.subcore_barrier()

    @pl.loop(0, nch)
    def _(k):
      p = lax.rem(k, 2)
      ichunk_wait(p)

      @pl.when(k + 1 < nch)
      def _():
        ichunk_start(k + 1, 1 - p)

      for b in range(nbuf):
        gather_start(p, b, b)

      @pl.loop(0, ngrp)
      def _(g):
        w0 = g * nbuf
        for b in range(nbuf):
          gather_wait(b)
          convert(b)
          scat_start(p, w0 + b, b)
        for b in range(nbuf):
          scat_wait(b)

          @pl.when(g + 1 < ngrp)
          def _():
            gather_start(p, w0 + nbuf + b, b)

    plsc.subcore_barrier()
    _readout(acc, out_hbm, c, s)

  return spmm


@functools.lru_cache(maxsize=None)
def _make_deg():
  nch = NWIN // CW
  ngrp = CW // NBUF
  scratch = [
      pltpu.VMEM((2, CW, B), jnp.int32),
      pltpu.VMEM_SHARED((N, 16), jnp.float32),
      pltpu.VMEM((ZR, 16), jnp.float32),
      pltpu.VMEM((B, 16), jnp.float32),
  ] + [pltpu.SemaphoreType.DMA for _ in range(NBUF + 1)]

  @functools.partial(
      pl.kernel,
      out_type=jax.ShapeDtypeStruct((NC, N, 16), jnp.float32),
      mesh=_mesh(),
      compiler_params=pltpu.CompilerParams(use_tc_tiling_on_sc=False),
      scratch_types=scratch)
  def deg(dst_hbm, out_hbm, dstv, acc, zbuf, ones_v, *sems):
    ssem = sems[:NBUF]
    isem = sems[NBUF]
    c = lax.axis_index("c")
    s = lax.axis_index("s")
    wid = c * NS + s

    def ichunk_start(k, p):
      pltpu.async_copy(dst_hbm.at[wid, pl.ds(k * CW, CW)], dstv.at[p], isem)

    def ichunk_wait(p):
      pltpu.make_async_copy(dst_hbm.at[wid, pl.ds(0, CW)], dstv.at[p],
                            isem).wait()

    ichunk_start(0, 0)
    ov = jnp.ones((16,), jnp.float32)

    @pl.loop(0, B)
    def _(r):
      ones_v[r, pl.ds(0, 16)] = ov

    _zero_acc(zbuf, acc, s, 16)
    plsc.subcore_barrier()

    @pl.loop(0, nch)
    def _(k):
      p = lax.rem(k, 2)
      ichunk_wait(p)

      @pl.when(k + 1 < nch)
      def _():
        ichunk_start(k + 1, 1 - p)

      @pl.loop(0, ngrp)
      def _(g):
        for b in range(NBUF):
          pltpu.async_copy(ones_v, acc.at[dstv.at[p, g * NBUF + b]], ssem[b],
                           add=True)
        for b in range(NBUF):
          pltpu.make_async_copy(ones_v, acc.at[dstv.at[0, 0]], ssem[b]).wait()

    plsc.subcore_barrier()
    _readout(acc, out_hbm, c, s)

  return deg


def _dinv_of(degp_ref):
  deg = degp_ref[0, :, 0] + degp_ref[1, :, 0] + 1.0
  return lax.rsqrt(deg)


FH = HID // NC


def _split_store(out_ref, h):
  out_ref[0] = h[:, :FH]
  out_ref[1] = h[:, FH:]


def _dual_store(hsf_ref, hsb_ref, hs_tau, hs_nat):
  _split_store(hsf_ref, hs_tau)
  hb = hs_nat.astype(jnp.bfloat16)
  hsb_ref[0] = hb[:, :FH]
  hsb_ref[1] = hb[:, FH:]


def _tc1_body(x_ref, w1_ref, w1t_ref, degp_ref, hsf_ref, hsb_ref):
  dinv = _dinv_of(degp_ref)[:, None]
  x = x_ref[...]
  h = jnp.dot(x, w1_ref[...], preferred_element_type=jnp.float32)
  ht = jnp.dot(x, w1t_ref[...], preferred_element_type=jnp.float32)
  _dual_store(hsf_ref, hsb_ref, ht * dinv, h * dinv)


def _tc_mid_body(p_ref, hs_ref, degp_ref, w_ref, wt_ref, b_ref, *out_refs,
                 split):
  dinv = _dinv_of(degp_ref)[:, None]
  agg = jnp.concatenate([p_ref[0] + hs_ref[0], p_ref[1] + hs_ref[1]], axis=1)
  h = jnp.maximum(dinv * agg + b_ref[...], 0.0)
  hs = jnp.dot(h, w_ref[...], preferred_element_type=jnp.float32)
  if split:
    ht = jnp.dot(h, wt_ref[...], preferred_element_type=jnp.float32)
    _dual_store(out_refs[0], out_refs[1], ht * dinv, hs * dinv)
  else:
    out_refs[0][...] = hs * dinv


def _tc4_body(p_ref, hs_ref, degp_ref, b3_ref, out_ref):
  dinv = _dinv_of(degp_ref)
  agg = dinv[:, None] * (p_ref[0] + p_ref[1] + hs_ref[...])
  out_ref[...] = agg[:, :OUT_C] + b3_ref[...]


def _deg_spec():
  return pl.BlockSpec((2, BR, 16), lambda i: (0, i, 0))


def _half_spec():
  return pl.BlockSpec((2, BR, FH), lambda i: (0, i, 0))


_DUAL_SPECS = [_half_spec(), _half_spec()]
_DUAL_SHAPES = [jax.ShapeDtypeStruct((NC, N, FH), jnp.float32),
                jax.ShapeDtypeStruct((NC, N, FH), jnp.bfloat16)]

_tc1 = pl.pallas_call(
    _tc1_body,
    grid=(GR,),
    in_specs=[
        pl.BlockSpec((BR, IN_C), lambda i: (i, 0)),
        pl.BlockSpec((IN_C, HID), lambda i: (0, 0)),
        pl.BlockSpec((IN_C, HID), lambda i: (0, 0)),
        _deg_spec(),
    ],
    out_specs=_DUAL_SPECS,
    out_shape=_DUAL_SHAPES)


def _tc_mid(FW, split):
  return pl.pallas_call(
      functools.partial(_tc_mid_body, split=split),
      grid=(GR,),
      in_specs=[
          _half_spec(),
          _half_spec(),
          _deg_spec(),
          pl.BlockSpec((HID, FW), lambda i: (0, 0)),
          pl.BlockSpec((HID, FW), lambda i: (0, 0)),
          pl.BlockSpec((1, HID), lambda i: (0, 0)),
      ],
      out_specs=_DUAL_SPECS if split else pl.BlockSpec(
          (BR, FW), lambda i: (i, 0)),
      out_shape=_DUAL_SHAPES if split else jax.ShapeDtypeStruct(
          (N, FW), jnp.float32))


_tc2 = _tc_mid(HID, split=True)
_tc3 = _tc_mid(16, split=False)

_tc4 = pl.pallas_call(
    _tc4_body,
    grid=(GR,),
    in_specs=[
        pl.BlockSpec((2, BR, 16), lambda i: (0, i, 0)),
        pl.BlockSpec((BR, 16), lambda i: (i, 0)),
        _deg_spec(),
        pl.BlockSpec((1, OUT_C), lambda i: (0, 0)),
    ],
    out_specs=pl.BlockSpec((BR, OUT_C), lambda i: (i, 0)),
    out_shape=jax.ShapeDtypeStruct((N, OUT_C), jnp.float32))


def kernel(x, edge_index, W1, b1, W2, b2, W3, b3):
  src_e = edge_index[0].reshape(NW, NWIN, B)
  dst_e = edge_index[1].reshape(NW, NWIN, B)
  src_f = edge_index[0].reshape(NS, E // NS // B, B)
  dst_f = edge_index[1].reshape(NS, E // NS // B, B)

  spmm_hid = _make_spmm(True)
  spmm_16 = _make_spmm(False)

  W1t = W1[:, TAU]
  W2p = W2[TAU, :]
  W2pt = W2p[:, TAU]
  b1p = b1[TAU].reshape(1, HID)
  W3pp = jnp.pad(W3, ((0, 0), (0, 16 - OUT_C)))[TAU, :]
  b2p = b2[TAU].reshape(1, HID)

  degp = _make_deg()(dst_e)
  hs1f, hs1b = _tc1(x, W1, W1t, degp)
  p1 = spmm_hid(hs1b, src_f, dst_f)
  hs2f, hs2b = _tc2(p1, hs1f, degp, W2p, W2pt, b1p)
  p2 = spmm_hid(hs2b, src_f, dst_f)
  hs3 = _tc3(p2, hs2f, degp, W3pp, W3pp, b2p)
  p3 = spmm_16(hs3, src_e, dst_e)
  out = _tc4(p3, hs3, degp, b3.reshape(1, OUT_C))
  return out


# speedup vs baseline: 1.9556x; 1.9556x over previous
import functools

import jax
import jax.numpy as jnp
from jax import lax
from jax.experimental import pallas as pl
from jax.experimental.pallas import tpu as pltpu
from jax.experimental.pallas import tpu_sc as plsc

N = 10000
E = 640000
IN_C = 116
HID = 128
OUT_C = 2

NC, NS = 2, 16
NW = NC * NS
EPW = E // NW
B = 80
NWIN = EPW // B
NBUF = 5
CW = 50
ROWS_BIG = 640
ROWS_LAST = N - 15 * ROWS_BIG
ZR = 80

BR = 2000
GR = N // BR

@functools.lru_cache(maxsize=None)
def _mesh():
  return plsc.VectorSubcoreMesh(
      core_axis_name="c", subcore_axis_name="s", num_cores=NC, num_subcores=NS)


def _zero_fill(zbuf, F):
  zv = jnp.zeros((16,), jnp.float32)

  @pl.loop(0, ZR)
  def _(r):
    for j in range(F // 16):
      zbuf[r, pl.ds(j * 16, 16)] = zv


def _zero_acc(zbuf, acc, s, F):
  _zero_fill(zbuf, F)
  nz = jnp.where(s == NS - 1, ROWS_LAST // ZR, ROWS_BIG // ZR)

  @pl.loop(0, nz)
  def _(k):
    pltpu.sync_copy(zbuf, acc.at[pl.ds(s * ROWS_BIG + k * ZR, ZR)])


def _readout(acc, out_hbm, c, s):
  @pl.when(s < NS - 1)
  def _():
    pltpu.sync_copy(acc.at[pl.ds(s * ROWS_BIG, ROWS_BIG)],
                    out_hbm.at[c, pl.ds(s * ROWS_BIG, ROWS_BIG)])

  @pl.when(s == NS - 1)
  def _():
    pltpu.sync_copy(acc.at[pl.ds((NS - 1) * ROWS_BIG, ROWS_LAST)],
                    out_hbm.at[c, pl.ds((NS - 1) * ROWS_BIG, ROWS_LAST)])


@functools.lru_cache(maxsize=None)
def _make_spmm(feature_split):
  if feature_split:
    fa = HID // NC
    nwin = E // NS // B
    cw, nbuf = 100, 10
  else:
    fa = 16
    nwin = E // NW // B
    cw, nbuf = 125, 5
  nch = nwin // cw
  ngrp = cw // nbuf

  scratch = [
      pltpu.VMEM((2, cw, B), jnp.int32),
      pltpu.VMEM((2, cw, B), jnp.int32),
      pltpu.VMEM_SHARED((N, fa), jnp.float32),
      pltpu.VMEM((ZR, fa), jnp.float32),
  ]
  scratch += [pltpu.VMEM((B, fa), jnp.float32) for _ in range(nbuf)]
  scratch += [pltpu.SemaphoreType.DMA for _ in range(2 * nbuf + 1)]

  @functools.partial(
      pl.kernel,
      out_type=jax.ShapeDtypeStruct((NC, N, fa), jnp.float32),
      mesh=_mesh(),
      compiler_params=pltpu.CompilerParams(use_tc_tiling_on_sc=False),
      scratch_types=scratch)
  def spmm(hs_hbm, src_hbm, dst_hbm, out_hbm, srcv, dstv, acc, zbuf, *rest):
    rows = rest[:nbuf]
    gsem = rest[nbuf:2 * nbuf]
    ssem = rest[2 * nbuf:3 * nbuf]
    isem = rest[3 * nbuf]
    c = lax.axis_index("c")
    s = lax.axis_index("s")
    iw = s if feature_split else c * NS + s
    table = hs_hbm.at[c] if feature_split else hs_hbm

    def ichunk_start(k, p):
      pltpu.async_copy(src_hbm.at[iw, pl.ds(k * cw, cw)], srcv.at[p], isem)
      pltpu.async_copy(dst_hbm.at[iw, pl.ds(k * cw, cw)], dstv.at[p], isem)

    def ichunk_wait(p):
      pltpu.make_async_copy(src_hbm.at[iw, pl.ds(0, cw)], srcv.at[p],
                            isem).wait()
      pltpu.make_async_copy(dst_hbm.at[iw, pl.ds(0, cw)], dstv.at[p],
                            isem).wait()

    def gather_start(p, w, b):
      pltpu.async_copy(table.at[srcv.at[p, w]], rows[b], gsem[b])

    def gather_wait(b):
      pltpu.make_async_copy(table.at[srcv.at[0, 0]], rows[b], gsem[b]).wait()

    def scat_start(p, w, b):
      pltpu.async_copy(rows[b], acc.at[dstv.at[p, w]], ssem[b], add=True)

    def scat_wait(b):
      pltpu.make_async_copy(rows[b], acc.at[dstv.at[0, 0]], ssem[b]).wait()

    ichunk_start(0, 0)
    _zero_acc(zbuf, acc, s, fa)
    plsc.subcore_barrier()

    @pl.loop(0, nch)
    def _(k):
      p = lax.rem(k, 2)
      ichunk_wait(p)

      @pl.when(k + 1 < nch)
      def _():
        ichunk_start(k + 1, 1 - p)

      for b in range(nbuf):
        gather_start(p, b, b)

      @pl.loop(0, ngrp)
      def _(g):
        w0 = g * nbuf
        for b in range(nbuf):
          gather_wait(b)
          scat_start(p, w0 + b, b)
        for b in range(nbuf):
          scat_wait(b)

          @pl.when(g + 1 < ngrp)
          def _():
            gather_start(p, w0 + nbuf + b, b)

    plsc.subcore_barrier()
    _readout(acc, out_hbm, c, s)

  return spmm


@functools.lru_cache(maxsize=None)
def _make_deg():
  nch = NWIN // CW
  ngrp = CW // NBUF
  scratch = [
      pltpu.VMEM((2, CW, B), jnp.int32),
      pltpu.VMEM_SHARED((N, 16), jnp.float32),
      pltpu.VMEM((ZR, 16), jnp.float32),
      pltpu.VMEM((B, 16), jnp.float32),
  ] + [pltpu.SemaphoreType.DMA for _ in range(NBUF + 1)]

  @functools.partial(
      pl.kernel,
      out_type=jax.ShapeDtypeStruct((NC, N, 16), jnp.float32),
      mesh=_mesh(),
      compiler_params=pltpu.CompilerParams(use_tc_tiling_on_sc=False),
      scratch_types=scratch)
  def deg(dst_hbm, out_hbm, dstv, acc, zbuf, ones_v, *sems):
    ssem = sems[:NBUF]
    isem = sems[NBUF]
    c = lax.axis_index("c")
    s = lax.axis_index("s")
    wid = c * NS + s

    def ichunk_start(k, p):
      pltpu.async_copy(dst_hbm.at[wid, pl.ds(k * CW, CW)], dstv.at[p], isem)

    def ichunk_wait(p):
      pltpu.make_async_copy(dst_hbm.at[wid, pl.ds(0, CW)], dstv.at[p],
                            isem).wait()

    ichunk_start(0, 0)
    ov = jnp.ones((16,), jnp.float32)

    @pl.loop(0, B)
    def _(r):
      ones_v[r, pl.ds(0, 16)] = ov

    _zero_acc(zbuf, acc, s, 16)
    plsc.subcore_barrier()

    @pl.loop(0, nch)
    def _(k):
      p = lax.rem(k, 2)
      ichunk_wait(p)

      @pl.when(k + 1 < nch)
      def _():
        ichunk_start(k + 1, 1 - p)

      @pl.loop(0, ngrp)
      def _(g):
        for b in range(NBUF):
          pltpu.async_copy(ones_v, acc.at[dstv.at[p, g * NBUF + b]], ssem[b],
                           add=True)
        for b in range(NBUF):
          pltpu.make_async_copy(ones_v, acc.at[dstv.at[0, 0]], ssem[b]).wait()

    plsc.subcore_barrier()
    _readout(acc, out_hbm, c, s)

  return deg


def _dinv_of(degp_ref):
  deg = degp_ref[0, :, 0] + degp_ref[1, :, 0] + 1.0
  return lax.rsqrt(deg)


FH = HID // NC


def _split_store(out_ref, h):
  out_ref[0] = h[:, :FH]
  out_ref[1] = h[:, FH:]


def _tc1_body(x_ref, w1_ref, degp_ref, hs1_ref):
  dinv = _dinv_of(degp_ref)
  h = jnp.dot(x_ref[...], w1_ref[...], preferred_element_type=jnp.float32)
  _split_store(hs1_ref, h * dinv[:, None])


def _tc_mid_body(p_ref, hs_ref, degp_ref, w_ref, b_ref, out_ref, *, split):
  dinv = _dinv_of(degp_ref)
  agg = jnp.concatenate([p_ref[0] + hs_ref[0], p_ref[1] + hs_ref[1]], axis=1)
  h = jnp.maximum(dinv[:, None] * agg + b_ref[...], 0.0)
  hs = jnp.dot(h, w_ref[...], preferred_element_type=jnp.float32)
  hs = hs * dinv[:, None]
  if split:
    _split_store(out_ref, hs)
  else:
    out_ref[...] = hs


def _tc4_body(p_ref, hs_ref, degp_ref, b3_ref, out_ref):
  dinv = _dinv_of(degp_ref)
  agg = dinv[:, None] * (p_ref[0] + p_ref[1] + hs_ref[...])
  out_ref[...] = agg[:, :OUT_C] + b3_ref[...]


def _deg_spec():
  return pl.BlockSpec((2, BR, 16), lambda i: (0, i, 0))


def _half_spec():
  return pl.BlockSpec((2, BR, FH), lambda i: (0, i, 0))


_tc1 = pl.pallas_call(
    _tc1_body,
    grid=(GR,),
    in_specs=[
        pl.BlockSpec((BR, IN_C), lambda i: (i, 0)),
        pl.BlockSpec((IN_C, HID), lambda i: (0, 0)),
        _deg_spec(),
    ],
    out_specs=_half_spec(),
    out_shape=jax.ShapeDtypeStruct((NC, N, FH), jnp.float32))


def _tc_mid(FW, split):
  return pl.pallas_call(
      functools.partial(_tc_mid_body, split=split),
      grid=(GR,),
      in_specs=[
          _half_spec(),
          _half_spec(),
          _deg_spec(),
          pl.BlockSpec((HID, FW), lambda i: (0, 0)),
          pl.BlockSpec((1, HID), lambda i: (0, 0)),
      ],
      out_specs=_half_spec() if split else pl.BlockSpec(
          (BR, FW), lambda i: (i, 0)),
      out_shape=jax.ShapeDtypeStruct(
          (NC, N, FH) if split else (N, FW), jnp.float32))


_tc2 = _tc_mid(HID, split=True)
_tc3 = _tc_mid(16, split=False)

_tc4 = pl.pallas_call(
    _tc4_body,
    grid=(GR,),
    in_specs=[
        pl.BlockSpec((2, BR, 16), lambda i: (0, i, 0)),
        pl.BlockSpec((BR, 16), lambda i: (i, 0)),
        _deg_spec(),
        pl.BlockSpec((1, OUT_C), lambda i: (0, 0)),
    ],
    out_specs=pl.BlockSpec((BR, OUT_C), lambda i: (i, 0)),
    out_shape=jax.ShapeDtypeStruct((N, OUT_C), jnp.float32))


def kernel(x, edge_index, W1, b1, W2, b2, W3, b3):
  src_e = edge_index[0].reshape(NW, NWIN, B)
  dst_e = edge_index[1].reshape(NW, NWIN, B)
  src_f = edge_index[0].reshape(NS, E // NS // B, B)
  dst_f = edge_index[1].reshape(NS, E // NS // B, B)

  spmm_hid = _make_spmm(True)
  spmm_16 = _make_spmm(False)

  degp = _make_deg()(dst_e)
  hs1 = _tc1(x, W1, degp)
  p1 = spmm_hid(hs1, src_f, dst_f)
  hs2 = _tc2(p1, hs1, degp, W2, b1.reshape(1, HID))
  p2 = spmm_hid(hs2, src_f, dst_f)
  W3p = jnp.pad(W3, ((0, 0), (0, 16 - OUT_C)))
  hs3 = _tc3(p2, hs2, degp, W3p, b2.reshape(1, HID))
  p3 = spmm_16(hs3, src_e, dst_e)
  out = _tc4(p3, hs3, degp, b3.reshape(1, OUT_C))
  return out


# speedup vs baseline: 1.9846x; 1.0148x over previous
import functools

import jax
import jax.numpy as jnp
from jax import lax
from jax.experimental import pallas as pl
from jax.experimental.pallas import tpu as pltpu
from jax.experimental.pallas import tpu_sc as plsc

N = 10000
E = 640000
IN_C = 116
HID = 128
OUT_C = 2

NC, NS = 2, 16
NW = NC * NS
EPW = E // NW
B = 80
NWIN = EPW // B
NBUF = 5
CW = 50
ROWS_BIG = 640
ROWS_LAST = N - 15 * ROWS_BIG
ZR = 80

BR = 2000
GR = N // BR

@functools.lru_cache(maxsize=None)
def _mesh():
  return plsc.VectorSubcoreMesh(
      core_axis_name="c", subcore_axis_name="s", num_cores=NC, num_subcores=NS)


def _zero_fill(zbuf, F):
  zv = jnp.zeros((16,), jnp.float32)

  @pl.loop(0, ZR)
  def _(r):
    for j in range(F // 16):
      zbuf[r, pl.ds(j * 16, 16)] = zv


def _zero_acc(zbuf, acc, s, F):
  _zero_fill(zbuf, F)
  nz = jnp.where(s == NS - 1, ROWS_LAST // ZR, ROWS_BIG // ZR)

  @pl.loop(0, nz)
  def _(k):
    pltpu.sync_copy(zbuf, acc.at[pl.ds(s * ROWS_BIG + k * ZR, ZR)])


def _readout(acc, out_hbm, c, s):
  @pl.when(s < NS - 1)
  def _():
    pltpu.sync_copy(acc.at[pl.ds(s * ROWS_BIG, ROWS_BIG)],
                    out_hbm.at[c, pl.ds(s * ROWS_BIG, ROWS_BIG)])

  @pl.when(s == NS - 1)
  def _():
    pltpu.sync_copy(acc.at[pl.ds((NS - 1) * ROWS_BIG, ROWS_LAST)],
                    out_hbm.at[c, pl.ds((NS - 1) * ROWS_BIG, ROWS_LAST)])


@functools.lru_cache(maxsize=None)
def _make_spmm(feature_split):
  if feature_split:
    fa = HID // NC
    nwin = E // NS // B
    cw, nbuf = 100, 10
  else:
    fa = 16
    nwin = E // NW // B
    cw, nbuf = 50, 10
  nch = nwin // cw
  ngrp = cw // nbuf

  scratch = [
      pltpu.VMEM((2, cw, B), jnp.int32),
      pltpu.VMEM((2, cw, B), jnp.int32),
      pltpu.VMEM_SHARED((N, fa), jnp.float32),
      pltpu.VMEM((ZR, fa), jnp.float32),
  ]
  scratch += [pltpu.VMEM((B, fa), jnp.float32) for _ in range(nbuf)]
  scratch += [pltpu.SemaphoreType.DMA for _ in range(2 * nbuf + 1)]

  @functools.partial(
      pl.kernel,
      out_type=jax.ShapeDtypeStruct((NC, N, fa), jnp.float32),
      mesh=_mesh(),
      compiler_params=pltpu.CompilerParams(use_tc_tiling_on_sc=False),
      scratch_types=scratch)
  def spmm(hs_hbm, src_hbm, dst_hbm, out_hbm, srcv, dstv, acc, zbuf, *rest):
    rows = rest[:nbuf]
    gsem = rest[nbuf:2 * nbuf]
    ssem = rest[2 * nbuf:3 * nbuf]
    isem = rest[3 * nbuf]
    c = lax.axis_index("c")
    s = lax.axis_index("s")
    iw = s if feature_split else c * NS + s
    table = hs_hbm.at[c] if feature_split else hs_hbm

    def ichunk_start(k, p):
      pltpu.async_copy(src_hbm.at[iw, pl.ds(k * cw, cw)], srcv.at[p], isem)
      pltpu.async_copy(dst_hbm.at[iw, pl.ds(k * cw, cw)], dstv.at[p], isem)

    def ichunk_wait(p):
      pltpu.make_async_copy(src_hbm.at[iw, pl.ds(0, cw)], srcv.at[p],
                            isem).wait()
      pltpu.make_async_copy(dst_hbm.at[iw, pl.ds(0, cw)], dstv.at[p],
                            isem).wait()

    def gather_start(p, w, b):
      pltpu.async_copy(table.at[srcv.at[p, w]], rows[b], gsem[b])

    def gather_wait(b):
      pltpu.make_async_copy(table.at[srcv.at[0, 0]], rows[b], gsem[b]).wait()

    def scat_start(p, w, b):
      pltpu.async_copy(rows[b], acc.at[dstv.at[p, w]], ssem[b], add=True)

    def scat_wait(b):
      pltpu.make_async_copy(rows[b], acc.at[dstv.at[0, 0]], ssem[b]).wait()

    ichunk_start(0, 0)
    _zero_acc(zbuf, acc, s, fa)
    plsc.subcore_barrier()

    @pl.loop(0, nch)
    def _(k):
      p = lax.rem(k, 2)
      ichunk_wait(p)

      @pl.when(k + 1 < nch)
      def _():
        ichunk_start(k + 1, 1 - p)

      for b in range(nbuf):
        gather_start(p, b, b)

      @pl.loop(0, ngrp)
      def _(g):
        w0 = g * nbuf
        for b in range(nbuf):
          gather_wait(b)
          scat_start(p, w0 + b, b)
        for b in range(nbuf):
          scat_wait(b)

          @pl.when(g + 1 < ngrp)
          def _():
            gather_start(p, w0 + nbuf + b, b)

    plsc.subcore_barrier()
    _readout(acc, out_hbm, c, s)

  return spmm


@functools.lru_cache(maxsize=None)
def _make_deg():
  nch = NWIN // CW
  ngrp = CW // NBUF
  scratch = [
      pltpu.VMEM((2, CW, B), jnp.int32),
      pltpu.VMEM_SHARED((N, 16), jnp.float32),
      pltpu.VMEM((ZR, 16), jnp.float32),
      pltpu.VMEM((B, 16), jnp.float32),
  ] + [pltpu.SemaphoreType.DMA for _ in range(NBUF + 1)]

  @functools.partial(
      pl.kernel,
      out_type=jax.ShapeDtypeStruct((NC, N, 16), jnp.float32),
      mesh=_mesh(),
      compiler_params=pltpu.CompilerParams(use_tc_tiling_on_sc=False),
      scratch_types=scratch)
  def deg(dst_hbm, out_hbm, dstv, acc, zbuf, ones_v, *sems):
    ssem = sems[:NBUF]
    isem = sems[NBUF]
    c = lax.axis_index("c")
    s = lax.axis_index("s")
    wid = c * NS + s

    def ichunk_start(k, p):
      pltpu.async_copy(dst_hbm.at[wid, pl.ds(k * CW, CW)], dstv.at[p], isem)

    def ichunk_wait(p):
      pltpu.make_async_copy(dst_hbm.at[wid, pl.ds(0, CW)], dstv.at[p],
                            isem).wait()

    ichunk_start(0, 0)
    ov = jnp.ones((16,), jnp.float32)

    @pl.loop(0, B)
    def _(r):
      ones_v[r, pl.ds(0, 16)] = ov

    _zero_acc(zbuf, acc, s, 16)
    plsc.subcore_barrier()

    @pl.loop(0, nch)
    def _(k):
      p = lax.rem(k, 2)
      ichunk_wait(p)

      @pl.when(k + 1 < nch)
      def _():
        ichunk_start(k + 1, 1 - p)

      @pl.loop(0, ngrp)
      def _(g):
        for b in range(NBUF):
          pltpu.async_copy(ones_v, acc.at[dstv.at[p, g * NBUF + b]], ssem[b],
                           add=True)
        for b in range(NBUF):
          pltpu.make_async_copy(ones_v, acc.at[dstv.at[0, 0]], ssem[b]).wait()

    plsc.subcore_barrier()
    _readout(acc, out_hbm, c, s)

  return deg


def _dinv_of(degp_ref):
  deg = degp_ref[0, :, 0] + degp_ref[1, :, 0] + 1.0
  return lax.rsqrt(deg)


FH = HID // NC


def _split_store(out_ref, h):
  out_ref[0] = h[:, :FH]
  out_ref[1] = h[:, FH:]


def _tc1_body(x_ref, w1_ref, degp_ref, hs1_ref):
  dinv = _dinv_of(degp_ref)
  h = jnp.dot(x_ref[...], w1_ref[...], preferred_element_type=jnp.float32)
  _split_store(hs1_ref, h * dinv[:, None])


def _tc_mid_body(p_ref, hs_ref, degp_ref, w_ref, b_ref, out_ref, *, split):
  dinv = _dinv_of(degp_ref)
  agg = jnp.concatenate([p_ref[0] + hs_ref[0], p_ref[1] + hs_ref[1]], axis=1)
  h = jnp.maximum(dinv[:, None] * agg + b_ref[...], 0.0)
  hs = jnp.dot(h, w_ref[...], preferred_element_type=jnp.float32)
  hs = hs * dinv[:, None]
  if split:
    _split_store(out_ref, hs)
  else:
    out_ref[...] = hs


def _tc4_body(p_ref, hs_ref, degp_ref, b3_ref, out_ref):
  dinv = _dinv_of(degp_ref)
  agg = dinv[:, None] * (p_ref[0] + p_ref[1] + hs_ref[...])
  out_ref[...] = agg[:, :OUT_C] + b3_ref[...]


def _deg_spec():
  return pl.BlockSpec((2, BR, 16), lambda i: (0, i, 0))


def _half_spec():
  return pl.BlockSpec((2, BR, FH), lambda i: (0, i, 0))


_tc1 = pl.pallas_call(
    _tc1_body,
    grid=(GR,),
    in_specs=[
        pl.BlockSpec((BR, IN_C), lambda i: (i, 0)),
        pl.BlockSpec((IN_C, HID), lambda i: (0, 0)),
        _deg_spec(),
    ],
    out_specs=_half_spec(),
    out_shape=jax.ShapeDtypeStruct((NC, N, FH), jnp.float32))


def _tc_mid(FW, split):
  return pl.pallas_call(
      functools.partial(_tc_mid_body, split=split),
      grid=(GR,),
      in_specs=[
          _half_spec(),
          _half_spec(),
          _deg_spec(),
          pl.BlockSpec((HID, FW), lambda i: (0, 0)),
          pl.BlockSpec((1, HID), lambda i: (0, 0)),
      ],
      out_specs=_half_spec() if split else pl.BlockSpec(
          (BR, FW), lambda i: (i, 0)),
      out_shape=jax.ShapeDtypeStruct(
          (NC, N, FH) if split else (N, FW), jnp.float32))


_tc2 = _tc_mid(HID, split=True)
_tc3 = _tc_mid(16, split=False)

_tc4 = pl.pallas_call(
    _tc4_body,
    grid=(GR,),
    in_specs=[
        pl.BlockSpec((2, BR, 16), lambda i: (0, i, 0)),
        pl.BlockSpec((BR, 16), lambda i: (i, 0)),
        _deg_spec(),
        pl.BlockSpec((1, OUT_C), lambda i: (0, 0)),
    ],
    out_specs=pl.BlockSpec((BR, OUT_C), lambda i: (i, 0)),
    out_shape=jax.ShapeDtypeStruct((N, OUT_C), jnp.float32))


def kernel(x, edge_index, W1, b1, W2, b2, W3, b3):
  src_e = edge_index[0].reshape(NW, NWIN, B)
  dst_e = edge_index[1].reshape(NW, NWIN, B)
  src_f = edge_index[0].reshape(NS, E // NS // B, B)
  dst_f = edge_index[1].reshape(NS, E // NS // B, B)

  spmm_hid = _make_spmm(True)
  spmm_16 = _make_spmm(False)

  degp = _make_deg()(dst_e)
  hs1 = _tc1(x, W1, degp)
  p1 = spmm_hid(hs1, src_f, dst_f)
  hs2 = _tc2(p1, hs1, degp, W2, b1.reshape(1, HID))
  p2 = spmm_hid(hs2, src_f, dst_f)
  W3p = jnp.pad(W3, ((0, 0), (0, 16 - OUT_C)))
  hs3 = _tc3(p2, hs2, degp, W3p, b2.reshape(1, HID))
  p3 = spmm_16(hs3, src_e, dst_e)
  out = _tc4(p3, hs3, degp, b3.reshape(1, OUT_C))
  return out


# speedup vs baseline: 1.9871x; 1.0012x over previous
import functools

import jax
import jax.numpy as jnp
from jax import lax
from jax.experimental import pallas as pl
from jax.experimental.pallas import tpu as pltpu
from jax.experimental.pallas import tpu_sc as plsc

N = 10000
E = 640000
IN_C = 116
HID = 128
OUT_C = 2

NC, NS = 2, 16
NW = NC * NS
EPW = E // NW
B = 80
NWIN = EPW // B
NBUF = 10
CW = 50
ROWS_BIG = 640
ROWS_LAST = N - 15 * ROWS_BIG
ZR = 80

BR = 2000
GR = N // BR

@functools.lru_cache(maxsize=None)
def _mesh():
  return plsc.VectorSubcoreMesh(
      core_axis_name="c", subcore_axis_name="s", num_cores=NC, num_subcores=NS)


def _zero_fill(zbuf, F):
  zv = jnp.zeros((16,), jnp.float32)

  @pl.loop(0, ZR)
  def _(r):
    for j in range(F // 16):
      zbuf[r, pl.ds(j * 16, 16)] = zv


def _zero_acc(zbuf, acc, s, F):
  _zero_fill(zbuf, F)
  nz = jnp.where(s == NS - 1, ROWS_LAST // ZR, ROWS_BIG // ZR)

  @pl.loop(0, nz)
  def _(k):
    pltpu.sync_copy(zbuf, acc.at[pl.ds(s * ROWS_BIG + k * ZR, ZR)])


def _readout(acc, out_hbm, c, s):
  @pl.when(s < NS - 1)
  def _():
    pltpu.sync_copy(acc.at[pl.ds(s * ROWS_BIG, ROWS_BIG)],
                    out_hbm.at[c, pl.ds(s * ROWS_BIG, ROWS_BIG)])

  @pl.when(s == NS - 1)
  def _():
    pltpu.sync_copy(acc.at[pl.ds((NS - 1) * ROWS_BIG, ROWS_LAST)],
                    out_hbm.at[c, pl.ds((NS - 1) * ROWS_BIG, ROWS_LAST)])


@functools.lru_cache(maxsize=None)
def _make_spmm(feature_split):
  if feature_split:
    fa = HID // NC
    nwin = E // NS // B
    cw, nbuf = 100, 10
  else:
    fa = 16
    nwin = E // NW // B
    cw, nbuf = 50, 10
  nch = nwin // cw
  ngrp = cw // nbuf

  scratch = [
      pltpu.VMEM((2, cw, B), jnp.int32),
      pltpu.VMEM((2, cw, B), jnp.int32),
      pltpu.VMEM_SHARED((N, fa), jnp.float32),
      pltpu.VMEM((ZR, fa), jnp.float32),
  ]
  scratch += [pltpu.VMEM((B, fa), jnp.float32) for _ in range(nbuf)]
  scratch += [pltpu.SemaphoreType.DMA for _ in range(2 * nbuf + 1)]

  @functools.partial(
      pl.kernel,
      out_type=jax.ShapeDtypeStruct((NC, N, fa), jnp.float32),
      mesh=_mesh(),
      compiler_params=pltpu.CompilerParams(use_tc_tiling_on_sc=False),
      scratch_types=scratch)
  def spmm(hs_hbm, src_hbm, dst_hbm, out_hbm, srcv, dstv, acc, zbuf, *rest):
    rows = rest[:nbuf]
    gsem = rest[nbuf:2 * nbuf]
    ssem = rest[2 * nbuf:3 * nbuf]
    isem = rest[3 * nbuf]
    c = lax.axis_index("c")
    s = lax.axis_index("s")
    iw = s if feature_split else c * NS + s
    table = hs_hbm.at[c] if feature_split else hs_hbm

    def ichunk_start(k, p):
      pltpu.async_copy(src_hbm.at[iw, pl.ds(k * cw, cw)], srcv.at[p], isem)
      pltpu.async_copy(dst_hbm.at[iw, pl.ds(k * cw, cw)], dstv.at[p], isem)

    def ichunk_wait(p):
      pltpu.make_async_copy(src_hbm.at[iw, pl.ds(0, cw)], srcv.at[p],
                            isem).wait()
      pltpu.make_async_copy(dst_hbm.at[iw, pl.ds(0, cw)], dstv.at[p],
                            isem).wait()

    def gather_start(p, w, b):
      pltpu.async_copy(table.at[srcv.at[p, w]], rows[b], gsem[b])

    def gather_wait(b):
      pltpu.make_async_copy(table.at[srcv.at[0, 0]], rows[b], gsem[b]).wait()

    def scat_start(p, w, b):
      pltpu.async_copy(rows[b], acc.at[dstv.at[p, w]], ssem[b], add=True)

    def scat_wait(b):
      pltpu.make_async_copy(rows[b], acc.at[dstv.at[0, 0]], ssem[b]).wait()

    ichunk_start(0, 0)
    _zero_acc(zbuf, acc, s, fa)
    plsc.subcore_barrier()

    @pl.loop(0, nch)
    def _(k):
      p = lax.rem(k, 2)
      ichunk_wait(p)

      @pl.when(k + 1 < nch)
      def _():
        ichunk_start(k + 1, 1 - p)

      for b in range(nbuf):
        gather_start(p, b, b)

      @pl.loop(0, ngrp)
      def _(g):
        w0 = g * nbuf
        for b in range(nbuf):
          gather_wait(b)
          scat_start(p, w0 + b, b)
        for b in range(nbuf):
          scat_wait(b)

          @pl.when(g + 1 < ngrp)
          def _():
            gather_start(p, w0 + nbuf + b, b)

    plsc.subcore_barrier()
    _readout(acc, out_hbm, c, s)

  return spmm


@functools.lru_cache(maxsize=None)
def _make_deg():
  nch = NWIN // CW
  ngrp = CW // NBUF
  scratch = [
      pltpu.VMEM((2, CW, B), jnp.int32),
      pltpu.VMEM_SHARED((N, 16), jnp.float32),
      pltpu.VMEM((ZR, 16), jnp.float32),
      pltpu.VMEM((B, 16), jnp.float32),
  ] + [pltpu.SemaphoreType.DMA for _ in range(NBUF + 1)]

  @functools.partial(
      pl.kernel,
      out_type=jax.ShapeDtypeStruct((NC, N, 16), jnp.float32),
      mesh=_mesh(),
      compiler_params=pltpu.CompilerParams(use_tc_tiling_on_sc=False),
      scratch_types=scratch)
  def deg(dst_hbm, out_hbm, dstv, acc, zbuf, ones_v, *sems):
    ssem = sems[:NBUF]
    isem = sems[NBUF]
    c = lax.axis_index("c")
    s = lax.axis_index("s")
    wid = c * NS + s

    def ichunk_start(k, p):
      pltpu.async_copy(dst_hbm.at[wid, pl.ds(k * CW, CW)], dstv.at[p], isem)

    def ichunk_wait(p):
      pltpu.make_async_copy(dst_hbm.at[wid, pl.ds(0, CW)], dstv.at[p],
                            isem).wait()

    ichunk_start(0, 0)
    ov = jnp.ones((16,), jnp.float32)

    @pl.loop(0, B)
    def _(r):
      ones_v[r, pl.ds(0, 16)] = ov

    _zero_acc(zbuf, acc, s, 16)
    plsc.subcore_barrier()

    @pl.loop(0, nch)
    def _(k):
      p = lax.rem(k, 2)
      ichunk_wait(p)

      @pl.when(k + 1 < nch)
      def _():
        ichunk_start(k + 1, 1 - p)

      @pl.loop(0, ngrp)
      def _(g):
        for b in range(NBUF):
          pltpu.async_copy(ones_v, acc.at[dstv.at[p, g * NBUF + b]], ssem[b],
                           add=True)
        for b in range(NBUF):
          pltpu.make_async_copy(ones_v, acc.at[dstv.at[0, 0]], ssem[b]).wait()

    plsc.subcore_barrier()
    _readout(acc, out_hbm, c, s)

  return deg


def _dinv_of(degp_ref):
  deg = degp_ref[0, :, 0] + degp_ref[1, :, 0] + 1.0
  return lax.rsqrt(deg)


FH = HID // NC


def _split_store(out_ref, h):
  out_ref[0] = h[:, :FH]
  out_ref[1] = h[:, FH:]


def _tc1_body(x_ref, w1_ref, degp_ref, hs1_ref):
  dinv = _dinv_of(degp_ref)
  h = jnp.dot(x_ref[...], w1_ref[...], preferred_element_type=jnp.float32)
  _split_store(hs1_ref, h * dinv[:, None])


def _tc_mid_body(p_ref, hs_ref, degp_ref, w_ref, b_ref, out_ref, *, split):
  dinv = _dinv_of(degp_ref)
  agg = jnp.concatenate([p_ref[0] + hs_ref[0], p_ref[1] + hs_ref[1]], axis=1)
  h = jnp.maximum(dinv[:, None] * agg + b_ref[...], 0.0)
  hs = jnp.dot(h, w_ref[...], preferred_element_type=jnp.float32)
  hs = hs * dinv[:, None]
  if split:
    _split_store(out_ref, hs)
  else:
    out_ref[...] = hs


def _tc4_body(p_ref, hs_ref, degp_ref, b3_ref, out_ref):
  dinv = _dinv_of(degp_ref)
  agg = dinv[:, None] * (p_ref[0] + p_ref[1] + hs_ref[...])
  out_ref[...] = agg[:, :OUT_C] + b3_ref[...]


def _deg_spec():
  return pl.BlockSpec((2, BR, 16), lambda i: (0, i, 0))


def _half_spec():
  return pl.BlockSpec((2, BR, FH), lambda i: (0, i, 0))


_tc1 = pl.pallas_call(
    _tc1_body,
    grid=(GR,),
    in_specs=[
        pl.BlockSpec((BR, IN_C), lambda i: (i, 0)),
        pl.BlockSpec((IN_C, HID), lambda i: (0, 0)),
        _deg_spec(),
    ],
    out_specs=_half_spec(),
    out_shape=jax.ShapeDtypeStruct((NC, N, FH), jnp.float32))


def _tc_mid(FW, split):
  return pl.pallas_call(
      functools.partial(_tc_mid_body, split=split),
      grid=(GR,),
      in_specs=[
          _half_spec(),
          _half_spec(),
          _deg_spec(),
          pl.BlockSpec((HID, FW), lambda i: (0, 0)),
          pl.BlockSpec((1, HID), lambda i: (0, 0)),
      ],
      out_specs=_half_spec() if split else pl.BlockSpec(
          (BR, FW), lambda i: (i, 0)),
      out_shape=jax.ShapeDtypeStruct(
          (NC, N, FH) if split else (N, FW), jnp.float32))


_tc2 = _tc_mid(HID, split=True)
_tc3 = _tc_mid(16, split=False)

_tc4 = pl.pallas_call(
    _tc4_body,
    grid=(GR,),
    in_specs=[
        pl.BlockSpec((2, BR, 16), lambda i: (0, i, 0)),
        pl.BlockSpec((BR, 16), lambda i: (i, 0)),
        _deg_spec(),
        pl.BlockSpec((1, OUT_C), lambda i: (0, 0)),
    ],
    out_specs=pl.BlockSpec((BR, OUT_C), lambda i: (i, 0)),
    out_shape=jax.ShapeDtypeStruct((N, OUT_C), jnp.float32))


def kernel(x, edge_index, W1, b1, W2, b2, W3, b3):
  src_e = edge_index[0].reshape(NW, NWIN, B)
  dst_e = edge_index[1].reshape(NW, NWIN, B)
  src_f = edge_index[0].reshape(NS, E // NS // B, B)
  dst_f = edge_index[1].reshape(NS, E // NS // B, B)

  spmm_hid = _make_spmm(True)
  spmm_16 = _make_spmm(False)

  degp = _make_deg()(dst_e)
  hs1 = _tc1(x, W1, degp)
  p1 = spmm_hid(hs1, src_f, dst_f)
  hs2 = _tc2(p1, hs1, degp, W2, b1.reshape(1, HID))
  p2 = spmm_hid(hs2, src_f, dst_f)
  W3p = jnp.pad(W3, ((0, 0), (0, 16 - OUT_C)))
  hs3 = _tc3(p2, hs2, degp, W3p, b2.reshape(1, HID))
  p3 = spmm_16(hs3, src_e, dst_e)
  out = _tc4(p3, hs3, degp, b3.reshape(1, OUT_C))
  return out
